# baseline (speedup 1.0000x reference)
import functools

import jax
import jax.numpy as jnp
import numpy as np
from jax import lax
from jax.experimental import pallas as pl
from jax.experimental.pallas import tpu as pltpu
from jax.experimental.pallas import tpu_sc as plsc

N = 10000
D = 128
NC = 2
NS = 16
K = 128

RPT = 632
NACC = NS * RPT
DEGW = 128

_R0 = np.int32(0)
_R1 = np.int32(1)

_mesh = plsc.VectorSubcoreMesh(core_axis_name="c", subcore_axis_name="s")


def _make_deg_kernel(chunks):
    @functools.partial(
        pl.kernel,
        out_type=jax.ShapeDtypeStruct((NC, NACC, DEGW), jnp.float32),
        mesh=_mesh,
        scratch_types=[
            [pltpu.VMEM((2, K), jnp.int32) for _ in range(4)],
            pltpu.VMEM((K, DEGW), jnp.float32),
            pltpu.VMEM_SHARED((NACC, DEGW), jnp.float32),
            [pltpu.SemaphoreType.DMA for _ in range(4)],
            [pltpu.SemaphoreType.DMA for _ in range(2)],
        ],
    )
    def deg_kernel(eidx_hbm, zeros_hbm, ones_hbm, out_hbm,
                   eidx, ones_v, acc_s, isem, ssem):
        cid = lax.axis_index("c")
        sid = lax.axis_index("s")
        wid = cid * NS + sid
        pltpu.sync_copy(zeros_hbm.at[pl.ds(sid * RPT, RPT)],
                        acc_s.at[pl.ds(sid * RPT, RPT)])
        pltpu.sync_copy(ones_hbm, ones_v)
        plsc.subcore_barrier()
        base = wid * jnp.int32(chunks)

        def fire_idx(ib, c):
            pltpu.async_copy(eidx_hbm.at[c], eidx[ib], isem[ib])

        def wait_idx(ib):
            pltpu.make_async_copy(eidx_hbm.at[base], eidx[ib],
                                  isem[ib]).wait()

        def fire_scatter(ib, sb):
            pltpu.async_copy(ones_v, acc_s.at[eidx[ib].at[_R1]], ssem[sb],
                             add=True)

        def wait_scatter(ib, sb):
            pltpu.make_async_copy(ones_v, acc_s.at[eidx[ib].at[_R1]],
                                  ssem[sb]).wait()

        def process(c, ib, sb, first):
            if not first:
                wait_scatter((ib + 2) % 4, sb)
            fire_idx((ib + 2) % 4, c + 2)
            wait_idx(ib)
            fire_scatter(ib, sb)

        fire_idx(0, base)
        fire_idx(1, base + 1)
        for t in range(4):
            process(base + t, t, t % 2, t < 2)

        @pl.loop(jnp.int32(4), jnp.int32(chunks), step=jnp.int32(4))
        def _quad(j):
            c = base + j
            for t in range(4):
                process(c + t, t, t % 2, False)

        wait_scatter(2, 0)
        wait_scatter(3, 1)
        wait_idx(0)
        wait_idx(1)
        plsc.subcore_barrier()
        pltpu.sync_copy(acc_s.at[pl.ds(sid * RPT, RPT)],
                        out_hbm.at[cid, pl.ds(sid * RPT, RPT)])

    return deg_kernel


def _make_scatter_kernel(ch0, ch1):
    @functools.partial(
        pl.kernel,
        out_type=jax.ShapeDtypeStruct((NC, NACC, D), jnp.float32),
        mesh=_mesh,
        scratch_types=[
            [pltpu.VMEM((2, K), jnp.int32) for _ in range(4)],
            [pltpu.VMEM((K, D), jnp.float32) for _ in range(2)],
            pltpu.VMEM_SHARED((NACC, D), jnp.float32),
            [pltpu.SemaphoreType.DMA for _ in range(4)],
            [pltpu.SemaphoreType.DMA for _ in range(2)],
            [pltpu.SemaphoreType.DMA for _ in range(2)],
        ],
    )
    def scatter_kernel(g_hbm, eidx_hbm, zeros_hbm, out_hbm,
                       eidx, rows, acc_s, isem, gsem, ssem):
        cid = lax.axis_index("c")
        sid = lax.axis_index("s")
        pltpu.sync_copy(zeros_hbm.at[pl.ds(sid * RPT, RPT)],
                        acc_s.at[pl.ds(sid * RPT, RPT)])
        plsc.subcore_barrier()
        base = jnp.where(cid == 0, sid * jnp.int32(ch0),
                         jnp.int32(NS * ch0) + sid * jnp.int32(ch1))
        base = base.astype(jnp.int32)
        chunks = jnp.where(cid == 0, jnp.int32(ch0), jnp.int32(ch1))

        def fire_idx(t, c):
            ib = t % 4
            pltpu.async_copy(eidx_hbm.at[c], eidx[ib], isem[ib])

        def wait_idx(t):
            ib = t % 4
            pltpu.make_async_copy(eidx_hbm.at[base], eidx[ib],
                                  isem[ib]).wait()

        def fire_gather(t):
            ib, rb = t % 4, t % 2
            pltpu.async_copy(g_hbm.at[eidx[ib].at[_R0]], rows[rb], gsem[rb])

        def wait_gather(t):
            ib, rb = t % 4, t % 2
            pltpu.make_async_copy(g_hbm.at[eidx[ib].at[_R0]], rows[rb],
                                  gsem[rb]).wait()

        def fire_scatter(t):
            ib, rb = t % 4, t % 2
            pltpu.async_copy(rows[rb], acc_s.at[eidx[ib].at[_R1]], ssem[rb],
                             add=True)

        def wait_scatter(t):
            ib, rb = t % 4, t % 2
            pltpu.make_async_copy(rows[rb], acc_s.at[eidx[ib].at[_R1]],
                                  ssem[rb]).wait()

        def process(c, t, first):
            if t >= 2 or not first:
                wait_scatter(t - 2)
            fire_idx(t + 2, c + 2)
            wait_idx(t)
            fire_gather(t)
            if t >= 1 or not first:
                wait_gather(t - 1)
                fire_scatter(t - 1)

        for t in range(2):
            fire_idx(t, base + t)
        for t in range(4):
            process(base + t, t, True)

        @pl.loop(jnp.int32(4), chunks, step=jnp.int32(4))
        def _quad(j):
            c = base + j
            for t in range(4):
                process(c + t, t, False)

        wait_gather(-1)
        fire_scatter(-1)
        wait_scatter(-2)
        wait_scatter(-1)
        for t in range(2):
            wait_idx(t)
        plsc.subcore_barrier()
        pltpu.sync_copy(acc_s.at[pl.ds(sid * RPT, RPT)],
                        out_hbm.at[cid, pl.ds(sid * RPT, RPT)])

    return scatter_kernel


_RB = 1000


def _dinv_of(deg_ref):
    deg = deg_ref[0, :, 0:1] + deg_ref[1, :, 0:1] + 1.0
    return lax.rsqrt(deg)


def _tc1_body(x_ref, w_ref, deg_ref, o_ref):
    dinv = _dinv_of(deg_ref)
    h = jnp.dot(x_ref[...], w_ref[...], preferred_element_type=jnp.float32)
    o_ref[...] = h * dinv


def _tc2_body(g_ref, acc_ref, deg_ref, b_ref, w_ref, o_ref):
    dinv = _dinv_of(deg_ref)
    s = g_ref[...] + acc_ref[0] + acc_ref[1]
    z = jnp.maximum(dinv * s + b_ref[...], 0.0)
    h = jnp.dot(z, w_ref[...], preferred_element_type=jnp.float32)
    o_ref[...] = h * dinv


def _tc3_body(g_ref, acc_ref, deg_ref, b_ref, o_ref):
    dinv = _dinv_of(deg_ref)
    s = g_ref[...] + acc_ref[0] + acc_ref[1]
    o_ref[...] = dinv * s + b_ref[...]


_i0 = np.int32(0)
_row_spec = pl.BlockSpec((_RB, D), lambda i: (i, _i0))
_acc_spec = pl.BlockSpec((NC, _RB, D), lambda i: (_i0, i, _i0))
_deg_spec = pl.BlockSpec((NC, _RB, DEGW), lambda i: (_i0, i, _i0))
_w_spec = pl.BlockSpec((D, D), lambda i: (_i0, _i0))
_b_spec = pl.BlockSpec((1, D), lambda i: (_i0, _i0))
_grid = (N // _RB,)

_tc1 = pl.pallas_call(
    _tc1_body, grid=_grid,
    in_specs=[_row_spec, _w_spec, _deg_spec],
    out_specs=_row_spec,
    out_shape=jax.ShapeDtypeStruct((N, D), jnp.float32))

_tc2 = pl.pallas_call(
    _tc2_body, grid=_grid,
    in_specs=[_row_spec, _acc_spec, _deg_spec, _b_spec, _w_spec],
    out_specs=_row_spec,
    out_shape=jax.ShapeDtypeStruct((N, D), jnp.float32))

_tc3 = pl.pallas_call(
    _tc3_body, grid=_grid,
    in_specs=[_row_spec, _acc_spec, _deg_spec, _b_spec],
    out_specs=_row_spec,
    out_shape=jax.ShapeDtypeStruct((N, D), jnp.float32))


_SKEW = 0.70


def kernel(x, edge_index, W1, b1, W2, b2):
    E = edge_index.shape[1]
    chunks = -(-E // (NC * NS * K))
    chunks = (chunks + 3) // 4 * 4
    totch = NC * NS * chunks
    epad = totch * K - E
    per_sc = NC * chunks
    ch0 = max(8, min(per_sc - 8, int(round(per_sc * _SKEW / 8)) * 8))
    ch1 = per_sc - ch0

    src = edge_index[0].astype(jnp.int32)
    dst = edge_index[1].astype(jnp.int32)
    if epad:
        src = jnp.concatenate([src, jnp.zeros((epad,), jnp.int32)])
        dst = jnp.concatenate([dst, jnp.full((epad,), N, jnp.int32)])
    eidx = jnp.stack([src.reshape(totch, K), dst.reshape(totch, K)], axis=1)
    pad_chunk = jnp.tile(
        jnp.stack([jnp.zeros((1, K), jnp.int32),
                   jnp.full((1, K), N, jnp.int32)], axis=1), (4, 1, 1))
    eidx = jnp.concatenate([eidx, pad_chunk], axis=0)

    zeros_acc = jnp.zeros((NACC, D), jnp.float32)
    ones_deg = jnp.ones((K, DEGW), jnp.float32)

    deg_k = _make_deg_kernel(chunks)
    scat_k = _make_scatter_kernel(ch0, ch1)

    deg = deg_k(eidx, zeros_acc, ones_deg)

    b1r = b1.reshape(1, D).astype(jnp.float32)
    b2r = b2.reshape(1, D).astype(jnp.float32)

    g1 = _tc1(x, W1, deg)
    acc1 = scat_k(g1, eidx, zeros_acc)
    g2 = _tc2(g1, acc1, deg, b1r, W2)
    acc2 = scat_k(g2, eidx, zeros_acc)
    out = _tc3(g2, acc2, deg, b2r)
    return out


# speedup vs baseline: 1.0483x; 1.0483x over previous
import functools

import jax
import jax.numpy as jnp
import numpy as np
from jax import lax
from jax.experimental import pallas as pl
from jax.experimental.pallas import tpu as pltpu
from jax.experimental.pallas import tpu_sc as plsc

N = 10000
D = 128
NC = 2
NS = 16
K = 128

RPT = 632
NACC = NS * RPT
DEGW = 128

_R0 = np.int32(0)
_R1 = np.int32(1)

_mesh = plsc.VectorSubcoreMesh(core_axis_name="c", subcore_axis_name="s")


def _make_deg_kernel(chunks):
    @functools.partial(
        pl.kernel,
        out_type=jax.ShapeDtypeStruct((NC, NACC, DEGW), jnp.float32),
        mesh=_mesh,
        scratch_types=[
            [pltpu.VMEM((2, K), jnp.int32) for _ in range(4)],
            pltpu.VMEM((K, DEGW), jnp.float32),
            pltpu.VMEM_SHARED((NACC, DEGW), jnp.float32),
            [pltpu.SemaphoreType.DMA for _ in range(4)],
            [pltpu.SemaphoreType.DMA for _ in range(2)],
        ],
    )
    def deg_kernel(eidx_hbm, zeros_hbm, ones_hbm, out_hbm,
                   eidx, ones_v, acc_s, isem, ssem):
        cid = lax.axis_index("c")
        sid = lax.axis_index("s")
        wid = cid * NS + sid
        pltpu.sync_copy(zeros_hbm.at[pl.ds(sid * RPT, RPT)],
                        acc_s.at[pl.ds(sid * RPT, RPT)])
        pltpu.sync_copy(ones_hbm, ones_v)
        plsc.subcore_barrier()
        base = wid * jnp.int32(chunks)

        def fire_idx(ib, c):
            pltpu.async_copy(eidx_hbm.at[c], eidx[ib], isem[ib])

        def wait_idx(ib):
            pltpu.make_async_copy(eidx_hbm.at[base], eidx[ib],
                                  isem[ib]).wait()

        def fire_scatter(ib, sb):
            pltpu.async_copy(ones_v, acc_s.at[eidx[ib].at[_R1]], ssem[sb],
                             add=True)

        def wait_scatter(ib, sb):
            pltpu.make_async_copy(ones_v, acc_s.at[eidx[ib].at[_R1]],
                                  ssem[sb]).wait()

        def process(c, ib, sb, first):
            if not first:
                wait_scatter((ib + 2) % 4, sb)
            fire_idx((ib + 2) % 4, c + 2)
            wait_idx(ib)
            fire_scatter(ib, sb)

        fire_idx(0, base)
        fire_idx(1, base + 1)
        for t in range(4):
            process(base + t, t, t % 2, t < 2)

        @pl.loop(jnp.int32(4), jnp.int32(chunks), step=jnp.int32(4))
        def _quad(j):
            c = base + j
            for t in range(4):
                process(c + t, t, t % 2, False)

        wait_scatter(2, 0)
        wait_scatter(3, 1)
        wait_idx(0)
        wait_idx(1)
        plsc.subcore_barrier()
        pltpu.sync_copy(acc_s.at[pl.ds(sid * RPT, RPT)],
                        out_hbm.at[cid, pl.ds(sid * RPT, RPT)])

    return deg_kernel


def _make_scatter_kernel(ch0, ch1):
    @functools.partial(
        pl.kernel,
        out_type=jax.ShapeDtypeStruct((NC, NACC, D), jnp.float32),
        mesh=_mesh,
        scratch_types=[
            [pltpu.VMEM((2, K), jnp.int32) for _ in range(4)],
            [pltpu.VMEM((K, D), jnp.float32) for _ in range(2)],
            pltpu.VMEM_SHARED((NACC, D), jnp.float32),
            [pltpu.SemaphoreType.DMA for _ in range(4)],
            [pltpu.SemaphoreType.DMA for _ in range(2)],
            [pltpu.SemaphoreType.DMA for _ in range(2)],
        ],
    )
    def scatter_kernel(g_hbm, eidx_hbm, zeros_hbm, out_hbm,
                       eidx, rows, acc_s, isem, gsem, ssem):
        cid = lax.axis_index("c")
        sid = lax.axis_index("s")
        pltpu.sync_copy(zeros_hbm.at[pl.ds(sid * RPT, RPT)],
                        acc_s.at[pl.ds(sid * RPT, RPT)])
        plsc.subcore_barrier()
        base = jnp.where(cid == 0, sid * jnp.int32(ch0),
                         jnp.int32(NS * ch0) + sid * jnp.int32(ch1))
        base = base.astype(jnp.int32)
        chunks = jnp.where(cid == 0, jnp.int32(ch0), jnp.int32(ch1))

        def fire_idx(t, c):
            ib = t % 4
            pltpu.async_copy(eidx_hbm.at[c], eidx[ib], isem[ib])

        def wait_idx(t):
            ib = t % 4
            pltpu.make_async_copy(eidx_hbm.at[base], eidx[ib],
                                  isem[ib]).wait()

        def fire_gather(t):
            ib, rb = t % 4, t % 2
            pltpu.async_copy(g_hbm.at[eidx[ib].at[_R0]], rows[rb], gsem[rb])

        def wait_gather(t):
            ib, rb = t % 4, t % 2
            pltpu.make_async_copy(g_hbm.at[eidx[ib].at[_R0]], rows[rb],
                                  gsem[rb]).wait()

        def fire_scatter(t):
            ib, rb = t % 4, t % 2
            pltpu.async_copy(rows[rb], acc_s.at[eidx[ib].at[_R1]], ssem[rb],
                             add=True)

        def wait_scatter(t):
            ib, rb = t % 4, t % 2
            pltpu.make_async_copy(rows[rb], acc_s.at[eidx[ib].at[_R1]],
                                  ssem[rb]).wait()

        def process(c, t, first):
            if t >= 2 or not first:
                wait_scatter(t - 2)
            fire_idx(t + 2, c + 2)
            wait_idx(t)
            fire_gather(t)
            if t >= 1 or not first:
                wait_gather(t - 1)
                fire_scatter(t - 1)

        for t in range(2):
            fire_idx(t, base + t)
        for t in range(4):
            process(base + t, t, True)

        @pl.loop(jnp.int32(4), chunks, step=jnp.int32(4))
        def _quad(j):
            c = base + j
            for t in range(4):
                process(c + t, t, False)

        wait_gather(-1)
        fire_scatter(-1)
        wait_scatter(-2)
        wait_scatter(-1)
        for t in range(2):
            wait_idx(t)
        plsc.subcore_barrier()
        pltpu.sync_copy(acc_s.at[pl.ds(sid * RPT, RPT)],
                        out_hbm.at[cid, pl.ds(sid * RPT, RPT)])

    return scatter_kernel


_RB = 1000


def _dinv_of(deg_ref):
    deg = deg_ref[0, :, 0:1] + deg_ref[1, :, 0:1] + 1.0
    return lax.rsqrt(deg)


def _tc1_body(x_ref, w_ref, deg_ref, o_ref):
    dinv = _dinv_of(deg_ref)
    h = jnp.dot(x_ref[...], w_ref[...], preferred_element_type=jnp.float32)
    o_ref[...] = h * dinv


def _tc2_body(g_ref, acc_ref, deg_ref, b_ref, w_ref, o_ref):
    dinv = _dinv_of(deg_ref)
    s = g_ref[...] + acc_ref[0] + acc_ref[1]
    z = jnp.maximum(dinv * s + b_ref[...], 0.0)
    h = jnp.dot(z, w_ref[...], preferred_element_type=jnp.float32)
    o_ref[...] = h * dinv


def _tc3_body(g_ref, acc_ref, deg_ref, b_ref, o_ref):
    dinv = _dinv_of(deg_ref)
    s = g_ref[...] + acc_ref[0] + acc_ref[1]
    o_ref[...] = dinv * s + b_ref[...]


_i0 = np.int32(0)
_row_spec = pl.BlockSpec((_RB, D), lambda i: (i, _i0))
_acc_spec = pl.BlockSpec((NC, _RB, D), lambda i: (_i0, i, _i0))
_deg_spec = pl.BlockSpec((NC, _RB, DEGW), lambda i: (_i0, i, _i0))
_w_spec = pl.BlockSpec((D, D), lambda i: (_i0, _i0))
_b_spec = pl.BlockSpec((1, D), lambda i: (_i0, _i0))
_grid = (N // _RB,)

_tc1 = pl.pallas_call(
    _tc1_body, grid=_grid,
    in_specs=[_row_spec, _w_spec, _deg_spec],
    out_specs=_row_spec,
    out_shape=jax.ShapeDtypeStruct((N, D), jnp.float32))

_tc2 = pl.pallas_call(
    _tc2_body, grid=_grid,
    in_specs=[_row_spec, _acc_spec, _deg_spec, _b_spec, _w_spec],
    out_specs=_row_spec,
    out_shape=jax.ShapeDtypeStruct((N, D), jnp.float32))

_tc3 = pl.pallas_call(
    _tc3_body, grid=_grid,
    in_specs=[_row_spec, _acc_spec, _deg_spec, _b_spec],
    out_specs=_row_spec,
    out_shape=jax.ShapeDtypeStruct((N, D), jnp.float32))


_SKEW = 0.85


def kernel(x, edge_index, W1, b1, W2, b2):
    E = edge_index.shape[1]
    chunks = -(-E // (NC * NS * K))
    chunks = (chunks + 3) // 4 * 4
    totch = NC * NS * chunks
    epad = totch * K - E
    per_sc = NC * chunks
    ch0 = max(8, min(per_sc - 8, int(round(per_sc * _SKEW / 8)) * 8))
    ch1 = per_sc - ch0

    src = edge_index[0].astype(jnp.int32)
    dst = edge_index[1].astype(jnp.int32)
    if epad:
        src = jnp.concatenate([src, jnp.zeros((epad,), jnp.int32)])
        dst = jnp.concatenate([dst, jnp.full((epad,), N, jnp.int32)])
    eidx = jnp.stack([src.reshape(totch, K), dst.reshape(totch, K)], axis=1)
    pad_chunk = jnp.tile(
        jnp.stack([jnp.zeros((1, K), jnp.int32),
                   jnp.full((1, K), N, jnp.int32)], axis=1), (4, 1, 1))
    eidx = jnp.concatenate([eidx, pad_chunk], axis=0)

    zeros_acc = jnp.zeros((NACC, D), jnp.float32)
    ones_deg = jnp.ones((K, DEGW), jnp.float32)

    deg_k = _make_deg_kernel(chunks)
    scat_k = _make_scatter_kernel(ch0, ch1)

    deg = deg_k(eidx, zeros_acc, ones_deg)

    b1r = b1.reshape(1, D).astype(jnp.float32)
    b2r = b2.reshape(1, D).astype(jnp.float32)

    g1 = _tc1(x, W1, deg)
    acc1 = scat_k(g1, eidx, zeros_acc)
    g2 = _tc2(g1, acc1, deg, b1r, W2)
    acc2 = scat_k(g2, eidx, zeros_acc)
    out = _tc3(g2, acc2, deg, b2r)
    return out


# speedup vs baseline: 1.2003x; 1.1450x over previous
import functools

import jax
import jax.numpy as jnp
import numpy as np
from jax import lax
from jax.experimental import pallas as pl
from jax.experimental.pallas import tpu as pltpu
from jax.experimental.pallas import tpu_sc as plsc

N = 10000
D = 128
NC = 2
NS = 16
K = 128

RPT = 632
NACC = NS * RPT
DEGW = 128

_R0 = np.int32(0)
_R1 = np.int32(1)

_mesh = plsc.VectorSubcoreMesh(core_axis_name="c", subcore_axis_name="s")


def _make_deg_kernel(chunks):
    @functools.partial(
        pl.kernel,
        out_type=jax.ShapeDtypeStruct((NC, NACC, DEGW), jnp.float32),
        mesh=_mesh,
        scratch_types=[
            [pltpu.VMEM((2, K), jnp.int32) for _ in range(4)],
            pltpu.VMEM((K, DEGW), jnp.float32),
            pltpu.VMEM_SHARED((NACC, DEGW), jnp.float32),
            [pltpu.SemaphoreType.DMA for _ in range(4)],
            [pltpu.SemaphoreType.DMA for _ in range(2)],
        ],
    )
    def deg_kernel(eidx_hbm, zeros_hbm, ones_hbm, out_hbm,
                   eidx, ones_v, acc_s, isem, ssem):
        cid = lax.axis_index("c")
        sid = lax.axis_index("s")
        wid = cid * NS + sid
        pltpu.sync_copy(zeros_hbm.at[pl.ds(sid * RPT, RPT)],
                        acc_s.at[pl.ds(sid * RPT, RPT)])
        pltpu.sync_copy(ones_hbm, ones_v)
        plsc.subcore_barrier()
        base = wid * jnp.int32(chunks)

        def fire_idx(ib, c):
            pltpu.async_copy(eidx_hbm.at[c], eidx[ib], isem[ib])

        def wait_idx(ib):
            pltpu.make_async_copy(eidx_hbm.at[base], eidx[ib],
                                  isem[ib]).wait()

        def fire_scatter(ib, sb):
            pltpu.async_copy(ones_v, acc_s.at[eidx[ib].at[_R1]], ssem[sb],
                             add=True)

        def wait_scatter(ib, sb):
            pltpu.make_async_copy(ones_v, acc_s.at[eidx[ib].at[_R1]],
                                  ssem[sb]).wait()

        def process(c, ib, sb, first):
            if not first:
                wait_scatter((ib + 2) % 4, sb)
            fire_idx((ib + 2) % 4, c + 2)
            wait_idx(ib)
            fire_scatter(ib, sb)

        fire_idx(0, base)
        fire_idx(1, base + 1)
        for t in range(4):
            process(base + t, t, t % 2, t < 2)

        @pl.loop(jnp.int32(4), jnp.int32(chunks), step=jnp.int32(4))
        def _quad(j):
            c = base + j
            for t in range(4):
                process(c + t, t, t % 2, False)

        wait_scatter(2, 0)
        wait_scatter(3, 1)
        wait_idx(0)
        wait_idx(1)
        plsc.subcore_barrier()
        pltpu.sync_copy(acc_s.at[pl.ds(sid * RPT, RPT)],
                        out_hbm.at[cid, pl.ds(sid * RPT, RPT)])

    return deg_kernel


def _make_scatter_kernel(ch0, ch1):
    @functools.partial(
        pl.kernel,
        out_type=jax.ShapeDtypeStruct((NC, NACC, D), jnp.float32),
        mesh=_mesh,
        scratch_types=[
            [pltpu.VMEM((2, K), jnp.int32) for _ in range(4)],
            [pltpu.VMEM((K, D), jnp.float32) for _ in range(2)],
            pltpu.VMEM_SHARED((NACC, D), jnp.float32),
            [pltpu.SemaphoreType.DMA for _ in range(4)],
            [pltpu.SemaphoreType.DMA for _ in range(2)],
            [pltpu.SemaphoreType.DMA for _ in range(2)],
        ],
    )
    def scatter_kernel(g_hbm, eidx_hbm, zeros_hbm, out_hbm,
                       eidx, rows, acc_s, isem, gsem, ssem):
        cid = lax.axis_index("c")
        sid = lax.axis_index("s")
        pltpu.sync_copy(zeros_hbm.at[pl.ds(sid * RPT, RPT)],
                        acc_s.at[pl.ds(sid * RPT, RPT)])
        plsc.subcore_barrier()
        base = jnp.where(cid == 0, sid * jnp.int32(ch0),
                         jnp.int32(NS * ch0) + sid * jnp.int32(ch1))
        base = base.astype(jnp.int32)
        chunks = jnp.where(cid == 0, jnp.int32(ch0), jnp.int32(ch1))

        def fire_idx(t, c):
            ib = t % 4
            pltpu.async_copy(eidx_hbm.at[c], eidx[ib], isem[ib])

        def wait_idx(t):
            ib = t % 4
            pltpu.make_async_copy(eidx_hbm.at[base], eidx[ib],
                                  isem[ib]).wait()

        def fire_gather(t):
            ib, rb = t % 4, t % 2
            pltpu.async_copy(g_hbm.at[eidx[ib].at[_R0]], rows[rb], gsem[rb])

        def wait_gather(t):
            ib, rb = t % 4, t % 2
            pltpu.make_async_copy(g_hbm.at[eidx[ib].at[_R0]], rows[rb],
                                  gsem[rb]).wait()

        def fire_scatter(t):
            ib, rb = t % 4, t % 2
            pltpu.async_copy(rows[rb], acc_s.at[eidx[ib].at[_R1]], ssem[rb],
                             add=True)

        def wait_scatter(t):
            ib, rb = t % 4, t % 2
            pltpu.make_async_copy(rows[rb], acc_s.at[eidx[ib].at[_R1]],
                                  ssem[rb]).wait()

        def process(c, t, first):
            if t >= 2 or not first:
                wait_scatter(t - 2)
            fire_idx(t + 2, c + 2)
            wait_idx(t)
            fire_gather(t)
            if t >= 1 or not first:
                wait_gather(t - 1)
                fire_scatter(t - 1)

        for t in range(2):
            fire_idx(t, base + t)
        for t in range(4):
            process(base + t, t, True)

        @pl.loop(jnp.int32(4), chunks, step=jnp.int32(4))
        def _quad(j):
            c = base + j
            for t in range(4):
                process(c + t, t, False)

        wait_gather(-1)
        fire_scatter(-1)
        wait_scatter(-2)
        wait_scatter(-1)
        for t in range(2):
            wait_idx(t)
        plsc.subcore_barrier()
        pltpu.sync_copy(acc_s.at[pl.ds(sid * RPT, RPT)],
                        out_hbm.at[cid, pl.ds(sid * RPT, RPT)])

    return scatter_kernel


_RB = 1000


def _dinv_of(deg_ref):
    deg = deg_ref[0, :, 0:1] + deg_ref[1, :, 0:1] + 1.0
    return lax.rsqrt(deg)


def _tc1_body(x_ref, w_ref, deg_ref, o_ref):
    dinv = _dinv_of(deg_ref)
    h = jnp.dot(x_ref[...], w_ref[...], preferred_element_type=jnp.float32)
    o_ref[...] = h * dinv


def _tc2_body(g_ref, acc_ref, deg_ref, b_ref, w_ref, o_ref):
    dinv = _dinv_of(deg_ref)
    s = g_ref[...] + acc_ref[0] + acc_ref[1]
    z = jnp.maximum(dinv * s + b_ref[...], 0.0)
    h = jnp.dot(z, w_ref[...], preferred_element_type=jnp.float32)
    o_ref[...] = h * dinv


def _tc3_body(g_ref, acc_ref, deg_ref, b_ref, o_ref):
    dinv = _dinv_of(deg_ref)
    s = g_ref[...] + acc_ref[0] + acc_ref[1]
    o_ref[...] = dinv * s + b_ref[...]


_i0 = np.int32(0)
_row_spec = pl.BlockSpec((_RB, D), lambda i: (i, _i0))
_acc_spec = pl.BlockSpec((NC, _RB, D), lambda i: (_i0, i, _i0))
_deg_spec = pl.BlockSpec((NC, _RB, DEGW), lambda i: (_i0, i, _i0))
_w_spec = pl.BlockSpec((D, D), lambda i: (_i0, _i0))
_b_spec = pl.BlockSpec((1, D), lambda i: (_i0, _i0))
_grid = (N // _RB,)

_tc1 = pl.pallas_call(
    _tc1_body, grid=_grid,
    in_specs=[_row_spec, _w_spec, _deg_spec],
    out_specs=_row_spec,
    out_shape=jax.ShapeDtypeStruct((N, D), jnp.float32))

_tc2 = pl.pallas_call(
    _tc2_body, grid=_grid,
    in_specs=[_row_spec, _acc_spec, _deg_spec, _b_spec, _w_spec],
    out_specs=_row_spec,
    out_shape=jax.ShapeDtypeStruct((N, D), jnp.float32))

_tc3 = pl.pallas_call(
    _tc3_body, grid=_grid,
    in_specs=[_row_spec, _acc_spec, _deg_spec, _b_spec],
    out_specs=_row_spec,
    out_shape=jax.ShapeDtypeStruct((N, D), jnp.float32))


_SKEW = 0.95


def kernel(x, edge_index, W1, b1, W2, b2):
    E = edge_index.shape[1]
    chunks = -(-E // (NC * NS * K))
    chunks = (chunks + 3) // 4 * 4
    totch = NC * NS * chunks
    epad = totch * K - E
    per_sc = NC * chunks
    ch0 = max(8, min(per_sc - 8, int(round(per_sc * _SKEW / 8)) * 8))
    ch1 = per_sc - ch0

    src = edge_index[0].astype(jnp.int32)
    dst = edge_index[1].astype(jnp.int32)
    if epad:
        src = jnp.concatenate([src, jnp.zeros((epad,), jnp.int32)])
        dst = jnp.concatenate([dst, jnp.full((epad,), N, jnp.int32)])
    eidx = jnp.stack([src.reshape(totch, K), dst.reshape(totch, K)], axis=1)
    pad_chunk = jnp.tile(
        jnp.stack([jnp.zeros((1, K), jnp.int32),
                   jnp.full((1, K), N, jnp.int32)], axis=1), (4, 1, 1))
    eidx = jnp.concatenate([eidx, pad_chunk], axis=0)

    zeros_acc = jnp.zeros((NACC, D), jnp.float32)
    ones_deg = jnp.ones((K, DEGW), jnp.float32)

    deg_k = _make_deg_kernel(chunks)
    scat_k = _make_scatter_kernel(ch0, ch1)

    deg = deg_k(eidx, zeros_acc, ones_deg)

    b1r = b1.reshape(1, D).astype(jnp.float32)
    b2r = b2.reshape(1, D).astype(jnp.float32)

    g1 = _tc1(x, W1, deg)
    acc1 = scat_k(g1, eidx, zeros_acc)
    g2 = _tc2(g1, acc1, deg, b1r, W2)
    acc2 = scat_k(g2, eidx, zeros_acc)
    out = _tc3(g2, acc2, deg, b2r)
    return out
